# baseline (speedup 1.0000x reference)
import functools

import jax
import jax.numpy as jnp
from jax import lax
from jax.experimental import pallas as pl
from jax.experimental.pallas import tpu as pltpu
from jax.experimental.pallas import tpu_sc as plsc

B, S, V, H, P, T = 64, 512, 30522, 768, 512, 2
LN_EPS = 1e-12

N = B * S
NC, NS = 2, 16
NW = NC * NS

NSPLIT = 2
GB = B // NSPLIT
NG = N // NSPLIT
NPAIR = NG // 2
PPW = NPAIR // NW
KP = 32
NCHUNK = PPW // KP
HV = H // 16

NSEG = B // (GB // 2)
BBQ = 4


def _sc_body(ids_hbm, table_hbm, stag_hbm, idx_v, a0, a1, b0, b1, outb,
             sem_a, sem_b, sem_o):
    wid = lax.axis_index("s") * NC + lax.axis_index("c")
    base = wid * PPW
    pltpu.sync_copy(ids_hbm.at[wid], idx_v)
    abufs = (a0, a1)
    bbufs = (b0, b1)
    cpa = pltpu.async_copy(table_hbm.at[idx_v.at[0, 0]], a0, sem_a)
    cpb = pltpu.async_copy(table_hbm.at[idx_v.at[1, 0]], b0, sem_b)
    op = None
    for c in range(NCHUNK):
        cur_a = abufs[c % 2]
        cur_b = bbufs[c % 2]
        cpa.wait()
        cpb.wait()
        if c + 1 < NCHUNK:
            cpa = pltpu.async_copy(
                table_hbm.at[idx_v.at[0, c + 1]], abufs[(c + 1) % 2], sem_a)
            cpb = pltpu.async_copy(
                table_hbm.at[idx_v.at[1, c + 1]], bbufs[(c + 1) % 2], sem_b)
        if op is not None:
            op.wait()

        @pl.loop(0, KP)
        def _pack_pair(p):
            for m in range(HV):
                a = cur_a[p, pl.ds(16 * m, 16)]
                b = cur_b[p, pl.ds(16 * m, 16)]
                y = plsc.pack(a, b, format=plsc.PackFormat.INTERLEAVED)
                outb[p, pl.ds(16 * m, 16)] = plsc.bitcast(y, jnp.float32)

        op = pltpu.async_copy(
            outb, stag_hbm.at[pl.ds(base + c * KP, KP)], sem_o)
    op.wait()


_sc_gather_pack = functools.partial(
    pl.kernel,
    mesh=plsc.VectorSubcoreMesh(core_axis_name="c", subcore_axis_name="s"),
    out_type=jax.ShapeDtypeStruct((NPAIR, H), jnp.float32),
    scratch_types=[
        pltpu.VMEM((2, NCHUNK, KP), jnp.int32),
        pltpu.VMEM((KP, H), jnp.float32),
        pltpu.VMEM((KP, H), jnp.float32),
        pltpu.VMEM((KP, H), jnp.float32),
        pltpu.VMEM((KP, H), jnp.float32),
        pltpu.VMEM((KP, H), jnp.float32),
        pltpu.SemaphoreType.DMA,
        pltpu.SemaphoreType.DMA,
        pltpu.SemaphoreType.DMA,
    ],
    compiler_params=pltpu.CompilerParams(needs_layout_passes=False),
)(_sc_body)


def _ln_math(x, tt, pos, t0, t1, gamma, beta):
    ttf = tt.astype(jnp.float32)[:, :, None]
    emb = x + pos + t0 + ttf * (t1 - t0)
    mu = jnp.mean(emb, axis=-1, keepdims=True)
    d = emb - mu
    var = jnp.mean(d * d, axis=-1, keepdims=True)
    rstd = lax.rsqrt(var + LN_EPS)
    return (d * rstd) * gamma + beta


def _tc_compute(stag_ref, tt_ref, pos_ref, type_ref, gamma_ref, beta_ref,
                out_ref):
    xw = lax.bitcast_convert_type(stag_ref[...], jnp.int32)
    lo = lax.bitcast_convert_type(xw << 16, jnp.float32)
    hi = lax.bitcast_convert_type(xw & jnp.int32(-65536), jnp.float32)
    pos = pos_ref[...][None]
    t0 = type_ref[0:1, :][None]
    t1 = type_ref[1:2, :][None]
    gamma = gamma_ref[...][None]
    beta = beta_ref[...][None]
    out_ref[0] = _ln_math(lo, tt_ref[0, :, 0, :], pos, t0, t1, gamma, beta)
    out_ref[1] = _ln_math(hi, tt_ref[1, :, 0, :], pos, t0, t1, gamma, beta)


def _tc_first_body(stag_ref, tt_ref, pos_ref, type_ref, gamma_ref, beta_ref,
                   out_ref):
    _tc_compute(stag_ref, tt_ref, pos_ref, type_ref, gamma_ref, beta_ref,
                out_ref)


def _tc_update_body(prev_ref, stag_ref, tt_ref, pos_ref, type_ref, gamma_ref,
                    beta_ref, out_ref):
    del prev_ref
    _tc_compute(stag_ref, tt_ref, pos_ref, type_ref, gamma_ref, beta_ref,
                out_ref)


_SMALL_SPECS = [
    pl.BlockSpec((S, H), lambda b: (0, 0)),
    pl.BlockSpec((T, H), lambda b: (0, 0)),
    pl.BlockSpec((1, H), lambda b: (0, 0)),
    pl.BlockSpec((1, H), lambda b: (0, 0)),
]


def _tc_ln_group(g, prev, stag, tt4, pos_emb, type_emb, gamma, beta):
    in_specs = [
        pl.BlockSpec((BBQ, S, H), lambda q: (q, 0, 0)),
        pl.BlockSpec((2, BBQ, 1, S), lambda q, g=g: (g, q, 0, 0)),
    ] + _SMALL_SPECS
    out_spec = pl.BlockSpec((2, BBQ, S, H), lambda q, g=g: (g, q, 0, 0))
    out_shape = jax.ShapeDtypeStruct((NSEG, GB // 2, S, H), jnp.float32)
    grid = ((GB // 2) // BBQ,)
    if g == 0:
        return pl.pallas_call(
            _tc_first_body,
            grid=grid,
            in_specs=in_specs,
            out_specs=out_spec,
            out_shape=out_shape,
        )(stag, tt4, pos_emb, type_emb, gamma, beta)
    return pl.pallas_call(
        _tc_update_body,
        grid=grid,
        in_specs=[pl.BlockSpec(memory_space=pl.ANY)] + in_specs,
        out_specs=out_spec,
        out_shape=out_shape,
        input_output_aliases={0: 0},
    )(prev, stag, tt4, pos_emb, type_emb, gamma, beta)


def kernel(input_ids, token_type_ids, word_embeddings, position_embeddings,
           token_type_embeddings, ln_gamma, ln_beta):
    ids_t = input_ids.reshape(NSPLIT, 2, NW, NCHUNK, KP).transpose(
        0, 2, 1, 3, 4)
    tt4 = token_type_ids.reshape(NSEG, GB // 2, 1, S)
    gamma = ln_gamma.reshape(1, H)
    beta = ln_beta.reshape(1, H)
    stags = [
        _sc_gather_pack(ids_t[g], word_embeddings).reshape(GB // 2, S, H)
        for g in range(NSPLIT)
    ]
    out = None
    for g in range(NSPLIT):
        out = _tc_ln_group(g, out, stags[g], tt4, position_embeddings,
                           token_type_embeddings, gamma, beta)
    return out.reshape(B, S, H)


# speedup vs baseline: 1.0281x; 1.0281x over previous
import functools

import jax
import jax.numpy as jnp
from jax import lax
from jax.experimental import pallas as pl
from jax.experimental.pallas import tpu as pltpu
from jax.experimental.pallas import tpu_sc as plsc

B, S, V, H, P, T = 64, 512, 30522, 768, 512, 2
LN_EPS = 1e-12

N = B * S
NC, NS = 2, 16
NW = NC * NS

NSPLIT = 1
GB = B // NSPLIT
NG = N // NSPLIT
NPAIR = NG // 2
PPW = NPAIR // NW
KP = 32
NCHUNK = PPW // KP
HV = H // 16

NSEG = B // (GB // 2)
BBQ = 4


def _sc_body(ids_hbm, table_hbm, stag_hbm, idx_v, a0, a1, b0, b1, outb,
             sem_a, sem_b, sem_o):
    wid = lax.axis_index("s") * NC + lax.axis_index("c")
    base = wid * PPW
    pltpu.sync_copy(ids_hbm.at[wid], idx_v)
    abufs = (a0, a1)
    bbufs = (b0, b1)
    KH = KP // 2

    def _start_gathers(c, buf_a, buf_b):
        return [
            pltpu.async_copy(table_hbm.at[idx_v.at[0, c, pl.ds(0, KH)]],
                             buf_a.at[pl.ds(0, KH)], sem_a),
            pltpu.async_copy(table_hbm.at[idx_v.at[0, c, pl.ds(KH, KH)]],
                             buf_a.at[pl.ds(KH, KH)], sem_a),
            pltpu.async_copy(table_hbm.at[idx_v.at[1, c, pl.ds(0, KH)]],
                             buf_b.at[pl.ds(0, KH)], sem_b),
            pltpu.async_copy(table_hbm.at[idx_v.at[1, c, pl.ds(KH, KH)]],
                             buf_b.at[pl.ds(KH, KH)], sem_b),
        ]

    cps = _start_gathers(0, a0, b0)
    op = None
    for c in range(NCHUNK):
        cur_a = abufs[c % 2]
        cur_b = bbufs[c % 2]
        for cp in cps:
            cp.wait()
        if c + 1 < NCHUNK:
            cps = _start_gathers(c + 1, abufs[(c + 1) % 2],
                                 bbufs[(c + 1) % 2])
        if op is not None:
            op.wait()

        @pl.loop(0, KP)
        def _pack_pair(p):
            for m in range(HV):
                a = cur_a[p, pl.ds(16 * m, 16)]
                b = cur_b[p, pl.ds(16 * m, 16)]
                y = plsc.pack(a, b, format=plsc.PackFormat.INTERLEAVED)
                outb[p, pl.ds(16 * m, 16)] = plsc.bitcast(y, jnp.float32)

        op = pltpu.async_copy(
            outb, stag_hbm.at[pl.ds(base + c * KP, KP)], sem_o)
    op.wait()


_sc_gather_pack = functools.partial(
    pl.kernel,
    mesh=plsc.VectorSubcoreMesh(core_axis_name="c", subcore_axis_name="s"),
    out_type=jax.ShapeDtypeStruct((NPAIR, H), jnp.float32),
    scratch_types=[
        pltpu.VMEM((2, NCHUNK, KP), jnp.int32),
        pltpu.VMEM((KP, H), jnp.float32),
        pltpu.VMEM((KP, H), jnp.float32),
        pltpu.VMEM((KP, H), jnp.float32),
        pltpu.VMEM((KP, H), jnp.float32),
        pltpu.VMEM((KP, H), jnp.float32),
        pltpu.SemaphoreType.DMA,
        pltpu.SemaphoreType.DMA,
        pltpu.SemaphoreType.DMA,
    ],
    compiler_params=pltpu.CompilerParams(needs_layout_passes=False),
)(_sc_body)


def _ln_math(x, tt, pos, t0, t1, gamma, beta):
    ttf = tt.astype(jnp.float32)[:, :, None]
    emb = x + pos + t0 + ttf * (t1 - t0)
    mu = jnp.mean(emb, axis=-1, keepdims=True)
    d = emb - mu
    var = jnp.mean(d * d, axis=-1, keepdims=True)
    rstd = lax.rsqrt(var + LN_EPS)
    return (d * rstd) * gamma + beta


def _tc_compute(stag_ref, tt_ref, pos_ref, type_ref, gamma_ref, beta_ref,
                out_ref):
    xw = lax.bitcast_convert_type(stag_ref[...], jnp.int32)
    lo = lax.bitcast_convert_type(xw << 16, jnp.float32)
    hi = lax.bitcast_convert_type(xw & jnp.int32(-65536), jnp.float32)
    pos = pos_ref[...][None]
    t0 = type_ref[0:1, :][None]
    t1 = type_ref[1:2, :][None]
    gamma = gamma_ref[...][None]
    beta = beta_ref[...][None]
    out_ref[0] = _ln_math(lo, tt_ref[0, :, 0, :], pos, t0, t1, gamma, beta)
    out_ref[1] = _ln_math(hi, tt_ref[1, :, 0, :], pos, t0, t1, gamma, beta)


def _tc_first_body(stag_ref, tt_ref, pos_ref, type_ref, gamma_ref, beta_ref,
                   out_ref):
    _tc_compute(stag_ref, tt_ref, pos_ref, type_ref, gamma_ref, beta_ref,
                out_ref)


def _tc_update_body(prev_ref, stag_ref, tt_ref, pos_ref, type_ref, gamma_ref,
                    beta_ref, out_ref):
    del prev_ref
    _tc_compute(stag_ref, tt_ref, pos_ref, type_ref, gamma_ref, beta_ref,
                out_ref)


_SMALL_SPECS = [
    pl.BlockSpec((S, H), lambda b: (0, 0)),
    pl.BlockSpec((T, H), lambda b: (0, 0)),
    pl.BlockSpec((1, H), lambda b: (0, 0)),
    pl.BlockSpec((1, H), lambda b: (0, 0)),
]


def _tc_ln_group(g, prev, stag, tt4, pos_emb, type_emb, gamma, beta):
    in_specs = [
        pl.BlockSpec((BBQ, S, H), lambda q: (q, 0, 0)),
        pl.BlockSpec((2, BBQ, 1, S), lambda q, g=g: (g, q, 0, 0)),
    ] + _SMALL_SPECS
    out_spec = pl.BlockSpec((2, BBQ, S, H), lambda q, g=g: (g, q, 0, 0))
    out_shape = jax.ShapeDtypeStruct((NSEG, GB // 2, S, H), jnp.float32)
    grid = ((GB // 2) // BBQ,)
    if g == 0:
        return pl.pallas_call(
            _tc_first_body,
            grid=grid,
            in_specs=in_specs,
            out_specs=out_spec,
            out_shape=out_shape,
            compiler_params=pltpu.CompilerParams(
                vmem_limit_bytes=110 * 1024 * 1024),
        )(stag, tt4, pos_emb, type_emb, gamma, beta)
    return pl.pallas_call(
        _tc_update_body,
        grid=grid,
        in_specs=[pl.BlockSpec(memory_space=pl.ANY)] + in_specs,
        out_specs=out_spec,
        out_shape=out_shape,
        input_output_aliases={0: 0},
        compiler_params=pltpu.CompilerParams(
            vmem_limit_bytes=110 * 1024 * 1024),
    )(prev, stag, tt4, pos_emb, type_emb, gamma, beta)


def kernel(input_ids, token_type_ids, word_embeddings, position_embeddings,
           token_type_embeddings, ln_gamma, ln_beta):
    ids_t = input_ids.reshape(NSPLIT, 2, NW, NCHUNK, KP).transpose(
        0, 2, 1, 3, 4)
    tt4 = token_type_ids.reshape(NSEG, GB // 2, 1, S)
    gamma = ln_gamma.reshape(1, H)
    beta = ln_beta.reshape(1, H)
    stags = [
        _sc_gather_pack(ids_t[g], word_embeddings).reshape(GB // 2, S, H)
        for g in range(NSPLIT)
    ]
    out = None
    for g in range(NSPLIT):
        out = _tc_ln_group(g, out, stags[g], tt4, position_embeddings,
                           token_type_embeddings, gamma, beta)
    return out.reshape(B, S, H)


# speedup vs baseline: 1.0317x; 1.0035x over previous
import functools

import jax
import jax.numpy as jnp
from jax import lax
from jax.experimental import pallas as pl
from jax.experimental.pallas import tpu as pltpu
from jax.experimental.pallas import tpu_sc as plsc

B, S, V, H, P, T = 64, 512, 30522, 768, 512, 2
LN_EPS = 1e-12

N = B * S
NC, NS = 2, 16
NW = NC * NS

NSPLIT = 1
GB = B // NSPLIT
NG = N // NSPLIT
NPAIR = NG // 2
PPW = NPAIR // NW
KP = 32
NCHUNK = PPW // KP
HV = H // 16

NSEG = B // (GB // 2)
BBQ = 4


def _sc_body(ids_hbm, table_hbm, stag_hbm, idx_v, a0, a1, b0, b1, outb,
             sem_a, sem_b, sem_o):
    wid = lax.axis_index("s") * NC + lax.axis_index("c")
    base = wid * PPW
    pltpu.sync_copy(ids_hbm.at[wid], idx_v)
    abufs = (a0, a1)
    bbufs = (b0, b1)
    cpa = pltpu.async_copy(table_hbm.at[idx_v.at[0, 0]], a0, sem_a)
    cpb = pltpu.async_copy(table_hbm.at[idx_v.at[1, 0]], b0, sem_b)
    op = None
    for c in range(NCHUNK):
        cur_a = abufs[c % 2]
        cur_b = bbufs[c % 2]
        cpa.wait()
        cpb.wait()
        if c + 1 < NCHUNK:
            cpa = pltpu.async_copy(
                table_hbm.at[idx_v.at[0, c + 1]], abufs[(c + 1) % 2], sem_a)
            cpb = pltpu.async_copy(
                table_hbm.at[idx_v.at[1, c + 1]], bbufs[(c + 1) % 2], sem_b)
        if op is not None:
            op.wait()

        @pl.loop(0, KP)
        def _pack_pair(p):
            for m in range(HV):
                a = cur_a[p, pl.ds(16 * m, 16)]
                b = cur_b[p, pl.ds(16 * m, 16)]
                y = plsc.pack(a, b, format=plsc.PackFormat.INTERLEAVED)
                outb[p, pl.ds(16 * m, 16)] = plsc.bitcast(y, jnp.float32)

        op = pltpu.async_copy(
            outb, stag_hbm.at[pl.ds(base + c * KP, KP)], sem_o)
    op.wait()


_sc_gather_pack = functools.partial(
    pl.kernel,
    mesh=plsc.VectorSubcoreMesh(core_axis_name="c", subcore_axis_name="s"),
    out_type=jax.ShapeDtypeStruct((NPAIR, H), jnp.float32),
    scratch_types=[
        pltpu.VMEM((2, NCHUNK, KP), jnp.int32),
        pltpu.VMEM((KP, H), jnp.float32),
        pltpu.VMEM((KP, H), jnp.float32),
        pltpu.VMEM((KP, H), jnp.float32),
        pltpu.VMEM((KP, H), jnp.float32),
        pltpu.VMEM((KP, H), jnp.float32),
        pltpu.SemaphoreType.DMA,
        pltpu.SemaphoreType.DMA,
        pltpu.SemaphoreType.DMA,
    ],
    compiler_params=pltpu.CompilerParams(needs_layout_passes=False),
)(_sc_body)


def _ln_math(x, tt, pos, t0, t1, gamma, beta):
    ttf = tt.astype(jnp.float32)[:, :, None]
    emb = x + pos + t0 + ttf * (t1 - t0)
    mu = jnp.mean(emb, axis=-1, keepdims=True)
    d = emb - mu
    var = jnp.mean(d * d, axis=-1, keepdims=True)
    rstd = lax.rsqrt(var + LN_EPS)
    return (d * rstd) * gamma + beta


def _tc_compute(stag_ref, tt_ref, pos_ref, type_ref, gamma_ref, beta_ref,
                out_ref):
    xw = lax.bitcast_convert_type(stag_ref[...], jnp.int32)
    lo = lax.bitcast_convert_type(xw << 16, jnp.float32)
    hi = lax.bitcast_convert_type(xw & jnp.int32(-65536), jnp.float32)
    pos = pos_ref[...][None]
    t0 = type_ref[0:1, :][None]
    t1 = type_ref[1:2, :][None]
    gamma = gamma_ref[...][None]
    beta = beta_ref[...][None]
    out_ref[0] = _ln_math(lo, tt_ref[0, :, 0, :], pos, t0, t1, gamma, beta)
    out_ref[1] = _ln_math(hi, tt_ref[1, :, 0, :], pos, t0, t1, gamma, beta)


def _tc_first_body(stag_ref, tt_ref, pos_ref, type_ref, gamma_ref, beta_ref,
                   out_ref):
    _tc_compute(stag_ref, tt_ref, pos_ref, type_ref, gamma_ref, beta_ref,
                out_ref)


def _tc_update_body(prev_ref, stag_ref, tt_ref, pos_ref, type_ref, gamma_ref,
                    beta_ref, out_ref):
    del prev_ref
    _tc_compute(stag_ref, tt_ref, pos_ref, type_ref, gamma_ref, beta_ref,
                out_ref)


_SMALL_SPECS = [
    pl.BlockSpec((S, H), lambda b: (0, 0)),
    pl.BlockSpec((T, H), lambda b: (0, 0)),
    pl.BlockSpec((1, H), lambda b: (0, 0)),
    pl.BlockSpec((1, H), lambda b: (0, 0)),
]


def _tc_ln_group(g, prev, stag, tt4, pos_emb, type_emb, gamma, beta):
    in_specs = [
        pl.BlockSpec((BBQ, S, H), lambda q: (q, 0, 0)),
        pl.BlockSpec((2, BBQ, 1, S), lambda q, g=g: (g, q, 0, 0)),
    ] + _SMALL_SPECS
    out_spec = pl.BlockSpec((2, BBQ, S, H), lambda q, g=g: (g, q, 0, 0))
    out_shape = jax.ShapeDtypeStruct((NSEG, GB // 2, S, H), jnp.float32)
    grid = ((GB // 2) // BBQ,)
    if g == 0:
        return pl.pallas_call(
            _tc_first_body,
            grid=grid,
            in_specs=in_specs,
            out_specs=out_spec,
            out_shape=out_shape,
            compiler_params=pltpu.CompilerParams(
                vmem_limit_bytes=110 * 1024 * 1024),
        )(stag, tt4, pos_emb, type_emb, gamma, beta)
    return pl.pallas_call(
        _tc_update_body,
        grid=grid,
        in_specs=[pl.BlockSpec(memory_space=pl.ANY)] + in_specs,
        out_specs=out_spec,
        out_shape=out_shape,
        input_output_aliases={0: 0},
        compiler_params=pltpu.CompilerParams(
            vmem_limit_bytes=110 * 1024 * 1024),
    )(prev, stag, tt4, pos_emb, type_emb, gamma, beta)


def kernel(input_ids, token_type_ids, word_embeddings, position_embeddings,
           token_type_embeddings, ln_gamma, ln_beta):
    ids_t = input_ids.reshape(NSPLIT, 2, NW, NCHUNK, KP).transpose(
        0, 2, 1, 3, 4)
    tt4 = token_type_ids.reshape(NSEG, GB // 2, 1, S)
    gamma = ln_gamma.reshape(1, H)
    beta = ln_beta.reshape(1, H)
    stags = [
        _sc_gather_pack(ids_t[g], word_embeddings).reshape(GB // 2, S, H)
        for g in range(NSPLIT)
    ]
    out = None
    for g in range(NSPLIT):
        out = _tc_ln_group(g, out, stags[g], tt4, position_embeddings,
                           token_type_embeddings, gamma, beta)
    return out.reshape(B, S, H)
